# baseline (speedup 1.0000x reference)
import functools

import jax
import jax.numpy as jnp
from jax.experimental import pallas as pl
from jax.experimental.pallas import tpu as pltpu

BLOCK_C = 2048
SPLIT = 4
N_BUF = 3
TAIL_SPLIT = 4


def _logits_kernel(inv_ref, mt_ref, ft_ref, o_ref, scratch, sems, *, steps, bc, n):
    i = pl.program_id(0)
    slot = jax.lax.rem(i, N_BUF)
    tail = n - (steps - 1) * bc

    def _copies(step, buf, total, split=SPLIT):
        rows = total // split
        return [
            pltpu.make_async_copy(
                scratch.at[buf, pl.ds(j * rows, rows), :],
                o_ref.at[pl.ds(step * bc + j * rows, rows), :],
                sems.at[buf, j],
            )
            for j in range(split)
        ]

    @pl.when(i >= N_BUF)
    def _wait_prev():
        for c in _copies(i - N_BUF, slot, bc):
            c.wait()

    scratch[slot] = jax.lax.dot_general(
        mt_ref[...].astype(jnp.bfloat16),
        (ft_ref[...] * inv_ref[0, 0]).astype(jnp.bfloat16),
        dimension_numbers=(((0,), (0,)), ((), ())),
        preferred_element_type=jnp.float32,
    )

    @pl.when(i < steps - 1)
    def _start_full():
        for c in _copies(i, slot, bc):
            c.start()

    @pl.when(i == steps - 1)
    def _start_tail():
        for c in _copies(i, slot, tail, split=TAIL_SPLIT):
            c.start()

    @pl.when(i == steps - 1)
    def _drain():
        for t in range(max(steps - N_BUF, 0), steps - 1):
            for c in _copies(t, jax.lax.rem(t, N_BUF), bc):
                c.wait()
        for c in _copies(steps - 1, slot, tail, split=TAIL_SPLIT):
            c.wait()


def kernel(feature, index, memory, temperature, momentum):
    b, k = feature.shape
    n = memory.shape[0]
    ft = feature.T
    mt = memory.T
    inv = jnp.reshape(1.0 / jnp.asarray(temperature, jnp.float32), (1, 1))
    steps = pl.cdiv(n, BLOCK_C)
    out_t = pl.pallas_call(
        functools.partial(_logits_kernel, steps=steps, bc=BLOCK_C, n=n),
        grid=(steps,),
        in_specs=[
            pl.BlockSpec(memory_space=pltpu.MemorySpace.SMEM),
            pl.BlockSpec((k, BLOCK_C), lambda i: (0, i)),
            pl.BlockSpec((k, b), lambda i: (0, 0)),
        ],
        out_specs=pl.BlockSpec(memory_space=pl.ANY),
        out_shape=jax.ShapeDtypeStruct((n, b), jnp.float32),
        scratch_shapes=[
            pltpu.VMEM((N_BUF, BLOCK_C, b), jnp.float32),
            pltpu.SemaphoreType.DMA((N_BUF, SPLIT)),
        ],
        compiler_params=pltpu.CompilerParams(
            dimension_semantics=("arbitrary",),
            allow_input_fusion=[False, False, False],
        ),
    )(inv, mt, ft)
    return out_t.T


# speedup vs baseline: 1.0000x; 1.0000x over previous
import functools

import jax
import jax.numpy as jnp
from jax.experimental import pallas as pl
from jax.experimental.pallas import tpu as pltpu

BLOCK_C = 2048
SPLIT = 4
N_BUF = 4
TAIL_SPLIT = 4


def _logits_kernel(inv_ref, mt_ref, ft_ref, o_ref, scratch, sems, *, steps, bc, n):
    i = pl.program_id(0)
    slot = jax.lax.rem(i, N_BUF)
    tail = n - (steps - 1) * bc

    def _copies(step, buf, total, split=SPLIT):
        rows = total // split
        return [
            pltpu.make_async_copy(
                scratch.at[buf, pl.ds(j * rows, rows), :],
                o_ref.at[pl.ds(step * bc + j * rows, rows), :],
                sems.at[buf, j],
            )
            for j in range(split)
        ]

    @pl.when(i >= N_BUF)
    def _wait_prev():
        for c in _copies(i - N_BUF, slot, bc):
            c.wait()

    scratch[slot] = jax.lax.dot_general(
        mt_ref[...].astype(jnp.bfloat16),
        (ft_ref[...] * inv_ref[0, 0]).astype(jnp.bfloat16),
        dimension_numbers=(((0,), (0,)), ((), ())),
        preferred_element_type=jnp.float32,
    )

    @pl.when(i < steps - 1)
    def _start_full():
        for c in _copies(i, slot, bc):
            c.start()

    @pl.when(i == steps - 1)
    def _start_tail():
        for c in _copies(i, slot, tail, split=TAIL_SPLIT):
            c.start()

    @pl.when(i == steps - 1)
    def _drain():
        for t in range(max(steps - N_BUF, 0), steps - 1):
            for c in _copies(t, jax.lax.rem(t, N_BUF), bc):
                c.wait()
        for c in _copies(steps - 1, slot, tail, split=TAIL_SPLIT):
            c.wait()


def kernel(feature, index, memory, temperature, momentum):
    b, k = feature.shape
    n = memory.shape[0]
    ft = feature.T
    mt = memory.T
    inv = jnp.reshape(1.0 / jnp.asarray(temperature, jnp.float32), (1, 1))
    steps = pl.cdiv(n, BLOCK_C)
    out_t = pl.pallas_call(
        functools.partial(_logits_kernel, steps=steps, bc=BLOCK_C, n=n),
        grid=(steps,),
        in_specs=[
            pl.BlockSpec(memory_space=pltpu.MemorySpace.SMEM),
            pl.BlockSpec((k, BLOCK_C), lambda i: (0, i)),
            pl.BlockSpec((k, b), lambda i: (0, 0)),
        ],
        out_specs=pl.BlockSpec(memory_space=pl.ANY),
        out_shape=jax.ShapeDtypeStruct((n, b), jnp.float32),
        scratch_shapes=[
            pltpu.VMEM((N_BUF, BLOCK_C, b), jnp.float32),
            pltpu.SemaphoreType.DMA((N_BUF, SPLIT)),
        ],
        compiler_params=pltpu.CompilerParams(
            dimension_semantics=("arbitrary",),
            allow_input_fusion=[False, False, False],
        ),
    )(inv, mt, ft)
    return out_t.T


# speedup vs baseline: 1.0011x; 1.0011x over previous
import functools

import jax
import jax.numpy as jnp
from jax.experimental import pallas as pl
from jax.experimental.pallas import tpu as pltpu

BLOCK_C = 2048
SPLIT = 4
N_BUF = 3
TAIL_SPLIT = 4


def _logits_kernel(inv_ref, mt_ref, ft_ref, o_ref, scratch, sems, *, steps, bc, n):
    i = pl.program_id(0)
    slot = jax.lax.rem(i, N_BUF)
    tail = n - (steps - 1) * bc

    def _copies(step, buf, total, split=SPLIT):
        rows = total // split
        return [
            pltpu.make_async_copy(
                scratch.at[buf, pl.ds(j * rows, rows), :],
                o_ref.at[pl.ds(step * bc + j * rows, rows), :],
                sems.at[buf, j],
            )
            for j in range(split)
        ]

    @pl.when(i >= N_BUF)
    def _wait_prev():
        for c in _copies(i - N_BUF, slot, bc):
            c.wait()

    scratch[slot] = jax.lax.dot_general(
        mt_ref[...].astype(jnp.bfloat16),
        (ft_ref[...] * inv_ref[0, 0]).astype(jnp.bfloat16),
        dimension_numbers=(((0,), (0,)), ((), ())),
        preferred_element_type=jnp.float32,
    )

    @pl.when(i < steps - 1)
    def _start_full():
        for c in _copies(i, slot, bc):
            c.start()

    @pl.when(i == steps - 1)
    def _start_tail():
        for c in _copies(i, slot, tail, split=TAIL_SPLIT):
            c.start()

    @pl.when(i == steps - 1)
    def _drain():
        for t in range(max(steps - N_BUF, 0), steps - 1):
            for c in _copies(t, jax.lax.rem(t, N_BUF), bc):
                c.wait()
        for c in _copies(steps - 1, slot, tail, split=TAIL_SPLIT):
            c.wait()


def kernel(feature, index, memory, temperature, momentum):
    b, k = feature.shape
    n = memory.shape[0]
    ft = feature.T
    mt = memory.T
    inv = jnp.reshape(1.0 / jnp.asarray(temperature, jnp.float32), (1, 1))
    steps = pl.cdiv(n, BLOCK_C)
    out_t = pl.pallas_call(
        functools.partial(_logits_kernel, steps=steps, bc=BLOCK_C, n=n),
        grid=(steps,),
        in_specs=[
            pl.BlockSpec(memory_space=pltpu.MemorySpace.SMEM),
            pl.BlockSpec((k, BLOCK_C), lambda i: (0, i)),
            pl.BlockSpec((k, b), lambda i: (0, 0)),
        ],
        out_specs=pl.BlockSpec(memory_space=pl.ANY),
        out_shape=jax.ShapeDtypeStruct((n, b), jnp.float32),
        scratch_shapes=[
            pltpu.VMEM((N_BUF, BLOCK_C, b), jnp.float32),
            pltpu.SemaphoreType.DMA((N_BUF, SPLIT)),
        ],
        compiler_params=pltpu.CompilerParams(
            dimension_semantics=("arbitrary",),
            allow_input_fusion=[False, False, False],
        ),
    )(inv, mt, ft)
    return out_t.T
